# baseline (speedup 1.0000x reference)
import functools

import jax
import jax.numpy as jnp
from jax import lax
from jax.experimental import pallas as pl
from jax.experimental.pallas import tpu as pltpu
from jax.experimental.pallas import tpu_sc as plsc

_A = 128
_DM = 4096
_RK = 64
_B = 64
_SPLIT = 64
_D = (_DM * _RK) // _SPLIT
_ROWS_IN = _A * _SPLIT
_ROWS_OUT = _B * _SPLIT
_NW = 32
_RPW = _ROWS_OUT // _NW
_CHUNK = 8
_NCHUNKS = _RPW // _CHUNK


_BG = 8
_NCOL = 16


def _body(w_hbm, idx_hbm, out_hbm,
          idx_v, buf0, buf1,
          sem_g0, sem_g1, sem_w0, sem_w1):
    wid = lax.axis_index("s") * 2 + lax.axis_index("c")
    bg = wid & 7
    c0 = (wid >> 3) * _NCOL

    pltpu.sync_copy(idx_hbm, idx_v)
    idx_slice = idx_v.at[pl.ds(bg * _BG, _BG)]

    bufs = (buf0, buf1)
    gsems = (sem_g0, sem_g1)
    wsems = (sem_w0, sem_w1)
    gathers = [None, None]
    writes = [None, None]

    gathers[0] = pltpu.async_copy(
        w_hbm.at[idx_slice, pl.ds(c0 * _D, _D)], bufs[0], gsems[0])
    for c in range(_NCOL):
        s = c % 2
        if c + 1 < _NCOL:
            s2 = (c + 1) % 2
            if writes[s2] is not None:
                writes[s2].wait()
            gathers[s2] = pltpu.async_copy(
                w_hbm.at[idx_slice, pl.ds((c0 + c + 1) * _D, _D)],
                bufs[s2], gsems[s2])
        gathers[s].wait()
        writes[s] = pltpu.async_copy(
            bufs[s],
            out_hbm.at[pl.ds(bg * _BG, _BG), pl.ds((c0 + c) * _D, _D)],
            wsems[s])
    writes[0].wait()
    writes[1].wait()


@functools.partial(jax.jit, static_argnums=())
def _sc_gather(wflat, ids):
    mesh = plsc.VectorSubcoreMesh(core_axis_name="c", subcore_axis_name="s")
    f = functools.partial(
        pl.kernel,
        mesh=mesh,
        out_type=jax.ShapeDtypeStruct((_B, _DM * _RK), jnp.float32),
        scratch_types=[
            pltpu.VMEM((_B,), jnp.int32),
            pltpu.VMEM((_CHUNK, _D), jnp.float32),
            pltpu.VMEM((_CHUNK, _D), jnp.float32),
            pltpu.SemaphoreType.DMA,
            pltpu.SemaphoreType.DMA,
            pltpu.SemaphoreType.DMA,
            pltpu.SemaphoreType.DMA,
        ],
    )(_body)
    return f(wflat, ids)


def kernel(weight, adapter_ids):
    wflat = weight.reshape(_A, _DM * _RK)
    ids = adapter_ids.astype(jnp.int32)
    out = _sc_gather(wflat, ids)
    return out.reshape(_B, _DM, _RK)


# speedup vs baseline: 6.7954x; 6.7954x over previous
import functools

import jax
import jax.numpy as jnp
from jax import lax
from jax.experimental import pallas as pl
from jax.experimental.pallas import tpu as pltpu
from jax.experimental.pallas import tpu_sc as plsc

_A = 128
_DM = 4096
_RK = 64
_B = 64
_NW = 32
_IDW = _B // _NW
_BANDS = _RK // 8
_CHALF = _DM // 2


def _body(w_hbm, idx_hbm, out_hbm,
          idx_v, buf0, buf1,
          sem_g0, sem_g1, sem_w0, sem_w1):
    wid = lax.axis_index("s") * 2 + lax.axis_index("c")

    pltpu.sync_copy(idx_hbm.at[wid], idx_v)

    def src(c):
        band, h = c // 2, c % 2
        return w_hbm.at[idx_v,
                        pl.ds(band * 8, 8),
                        pl.ds(h * _CHALF, _CHALF)]

    def dst(c):
        band, h = c // 2, c % 2
        return out_hbm.at[pl.ds(wid * _IDW, _IDW),
                          pl.ds(band * 8, 8),
                          pl.ds(h * _CHALF, _CHALF)]

    nchunks = _BANDS * 2
    bufs = (buf0, buf1)
    gsems = (sem_g0, sem_g1)
    wsems = (sem_w0, sem_w1)
    gathers = [None, None]
    writes = [None, None]

    gathers[0] = pltpu.async_copy(src(0), bufs[0], gsems[0])
    for c in range(nchunks):
        s = c % 2
        if c + 1 < nchunks:
            s2 = (c + 1) % 2
            if writes[s2] is not None:
                writes[s2].wait()
            gathers[s2] = pltpu.async_copy(src(c + 1), bufs[s2], gsems[s2])
        gathers[s].wait()
        writes[s] = pltpu.async_copy(bufs[s], dst(c), wsems[s])
    writes[0].wait()
    writes[1].wait()


@jax.jit
def _sc_gather(wv, idx2):
    mesh = plsc.VectorSubcoreMesh(core_axis_name="c", subcore_axis_name="s")
    f = functools.partial(
        pl.kernel,
        mesh=mesh,
        out_type=jax.ShapeDtypeStruct((_B, _RK, _DM), jnp.float32),
        scratch_types=[
            pltpu.VMEM((_IDW,), jnp.int32),
            pltpu.VMEM((_IDW, 8, _CHALF), jnp.float32),
            pltpu.VMEM((_IDW, 8, _CHALF), jnp.float32),
            pltpu.SemaphoreType.DMA,
            pltpu.SemaphoreType.DMA,
            pltpu.SemaphoreType.DMA,
            pltpu.SemaphoreType.DMA,
        ],
    )(_body)
    return f(wv, idx2)


def kernel(weight, adapter_ids):
    wv = jnp.swapaxes(weight, 1, 2)
    idx2 = adapter_ids.astype(jnp.int32).reshape(_NW, _IDW)
    out = _sc_gather(wv, idx2)
    return jnp.swapaxes(out, 1, 2)


# speedup vs baseline: 6.8905x; 1.0140x over previous
import functools

import jax
import jax.numpy as jnp
from jax import lax
from jax.experimental import pallas as pl
from jax.experimental.pallas import tpu as pltpu
from jax.experimental.pallas import tpu_sc as plsc

_A = 128
_DM = 4096
_RK = 64
_B = 64
_NW = 32
_IDW = _B // _NW
_BANDS = _RK // 8
_CHALF = _DM // 2


def _body(w_hbm, idx_hbm, out_hbm,
          idx_v, buf0, buf1, buf2,
          sem_g0, sem_g1, sem_g2, sem_w0, sem_w1, sem_w2):
    wid = lax.axis_index("s") * 2 + lax.axis_index("c")

    pltpu.sync_copy(idx_hbm.at[wid], idx_v)

    def src(c):
        band, h = c // 2, c % 2
        return w_hbm.at[idx_v,
                        pl.ds(band * 8, 8),
                        pl.ds(h * _CHALF, _CHALF)]

    def dst(c):
        band, h = c // 2, c % 2
        return out_hbm.at[pl.ds(wid * _IDW, _IDW),
                          pl.ds(band * 8, 8),
                          pl.ds(h * _CHALF, _CHALF)]

    nchunks = _BANDS * 2
    bufs = (buf0, buf1, buf2)
    gsems = (sem_g0, sem_g1, sem_g2)
    wsems = (sem_w0, sem_w1, sem_w2)
    nbuf = 3
    gathers = [None] * nbuf
    writes = [None] * nbuf

    gathers[0] = pltpu.async_copy(src(0), bufs[0], gsems[0])
    gathers[1] = pltpu.async_copy(src(1), bufs[1], gsems[1])
    for c in range(nchunks):
        s = c % nbuf
        nxt = c + 2
        if nxt < nchunks:
            sn = nxt % nbuf
            if writes[sn] is not None:
                writes[sn].wait()
            gathers[sn] = pltpu.async_copy(src(nxt), bufs[sn], gsems[sn])
        gathers[s].wait()
        writes[s] = pltpu.async_copy(bufs[s], dst(c), wsems[s])
    for s in range(nbuf):
        writes[s].wait()


@jax.jit
def _sc_gather(wv, idx2):
    mesh = plsc.VectorSubcoreMesh(core_axis_name="c", subcore_axis_name="s")
    f = functools.partial(
        pl.kernel,
        mesh=mesh,
        out_type=jax.ShapeDtypeStruct((_B, _RK, _DM), jnp.float32),
        scratch_types=[
            pltpu.VMEM((_IDW,), jnp.int32),
            pltpu.VMEM((_IDW, 8, _CHALF), jnp.float32),
            pltpu.VMEM((_IDW, 8, _CHALF), jnp.float32),
            pltpu.VMEM((_IDW, 8, _CHALF), jnp.float32),
            pltpu.SemaphoreType.DMA,
            pltpu.SemaphoreType.DMA,
            pltpu.SemaphoreType.DMA,
            pltpu.SemaphoreType.DMA,
            pltpu.SemaphoreType.DMA,
            pltpu.SemaphoreType.DMA,
        ],
    )(_body)
    return f(wv, idx2)


def kernel(weight, adapter_ids):
    wv = jnp.swapaxes(weight, 1, 2)
    idx2 = adapter_ids.astype(jnp.int32).reshape(_NW, _IDW)
    out = _sc_gather(wv, idx2)
    return jnp.swapaxes(out, 1, 2)
